# Initial kernel scaffold; baseline (speedup 1.0000x reference)
import jax
import jax.numpy as jnp
from jax.experimental import pallas as pl


def kernel(input_ids, table):
    raise NotImplementedError("write your pallas kernel here")


# speedup vs baseline: 1.2429x; 1.2429x over previous
import functools

import jax
import jax.numpy as jnp
from jax import lax
from jax.experimental import pallas as pl
from jax.experimental.pallas import tpu as pltpu
from jax.experimental.pallas import tpu_sc as plsc

_LANES = 16


@functools.cache
def _build_sc_gather(n_rows, vocab, d, seq_len):
    info = plsc.get_sparse_core_info()
    nc, ns = info.num_cores, info.num_subcores
    nw = nc * ns
    assert n_rows % nw == 0
    b_per_w = n_rows // nw
    chunk = 64
    assert b_per_w % chunk == 0
    n_chunks = b_per_w // chunk
    ts_steps = b_per_w // _LANES
    assert b_per_w % _LANES == 0 and b_per_w % seq_len == 0
    inv = 1.0 / float(seq_len - 1)

    mesh = plsc.VectorSubcoreMesh(core_axis_name="c", subcore_axis_name="s")

    @functools.partial(
        pl.kernel,
        mesh=mesh,
        out_type=[
            jax.ShapeDtypeStruct((n_rows, d), jnp.float32),
            jax.ShapeDtypeStruct((n_rows,), jnp.float32),
        ],
        scratch_types=[
            pltpu.VMEM((n_chunks, chunk), jnp.int32),
            pltpu.VMEM((chunk, d), jnp.float32),
            pltpu.VMEM((b_per_w,), jnp.float32),
            pltpu.SemaphoreType.DMA,
        ],
    )
    def sc_gather(ids_hbm, table_hbm, emb_out, ts_out, idx_v, buf, ts_v, gsem):
        wid = lax.axis_index("s") * nc + lax.axis_index("c")
        base = wid * b_per_w

        pltpu.sync_copy(ids_hbm.at[wid], idx_v)

        def chunk_body(j, carry):
            pltpu.async_copy(table_hbm.at[idx_v.at[j]], buf, gsem).wait()
            pltpu.sync_copy(buf, emb_out.at[pl.ds(base + j * chunk, chunk)])
            return carry

        lax.fori_loop(0, n_chunks, chunk_body, 0)

        def ts_body(i, carry):
            p = lax.iota(jnp.int32, _LANES) + i * _LANES
            val = lax.rem(p, seq_len).astype(jnp.float32) * inv
            ts_v[pl.ds(i * _LANES, _LANES)] = val
            return carry

        lax.fori_loop(0, ts_steps, ts_body, 0)
        pltpu.sync_copy(ts_v, ts_out.at[pl.ds(base, b_per_w)])

    return sc_gather


def kernel(input_ids, table):
    b, l = input_ids.shape
    vocab, d = table.shape
    n_rows = b * l
    nw = 32
    ids3d = input_ids.reshape(-1).astype(jnp.int32).reshape(nw, -1, 64)
    emb_flat, ts_flat = _build_sc_gather(n_rows, vocab, d, l)(ids3d, table)
    return emb_flat.reshape(b, l, d), ts_flat.reshape(b, l)


# speedup vs baseline: 1.2773x; 1.0277x over previous
import functools

import jax
import jax.numpy as jnp
from jax import lax
from jax.experimental import pallas as pl
from jax.experimental.pallas import tpu as pltpu
from jax.experimental.pallas import tpu_sc as plsc

_LANES = 16
_CHUNK = 40


@functools.cache
def _build_sc_gather(n_rows, vocab, d, seq_len):
    info = plsc.get_sparse_core_info()
    nc, ns = info.num_cores, info.num_subcores
    nw = nc * ns
    assert n_rows % nw == 0
    b_per_w = n_rows // nw
    chunk = _CHUNK
    assert b_per_w % chunk == 0
    n_chunks = b_per_w // chunk
    assert n_chunks % 2 == 0 and n_chunks >= 4
    ts_steps = b_per_w // _LANES
    assert b_per_w % _LANES == 0 and b_per_w % seq_len == 0
    inv = 1.0 / float(seq_len - 1)

    mesh = plsc.VectorSubcoreMesh(core_axis_name="c", subcore_axis_name="s")

    @functools.partial(
        pl.kernel,
        mesh=mesh,
        out_type=[
            jax.ShapeDtypeStruct((n_rows, d), jnp.float32),
            jax.ShapeDtypeStruct((n_rows,), jnp.float32),
        ],
        scratch_types=[
            pltpu.VMEM((n_chunks, chunk), jnp.int32),
            pltpu.VMEM((chunk, d), jnp.float32),
            pltpu.VMEM((chunk, d), jnp.float32),
            pltpu.VMEM((b_per_w,), jnp.float32),
            pltpu.SemaphoreType.DMA,
            pltpu.SemaphoreType.DMA,
            pltpu.SemaphoreType.DMA,
            pltpu.SemaphoreType.DMA,
        ],
    )
    def sc_gather(ids_hbm, table_hbm, emb_out, ts_out,
                  idx_v, buf_a, buf_b, ts_v, gsa, gsb, ssa, ssb):
        wid = lax.axis_index("s") * nc + lax.axis_index("c")
        base = wid * b_per_w

        pltpu.sync_copy(ids_hbm.at[wid], idx_v)

        def gather(j, buf, sem):
            return pltpu.make_async_copy(table_hbm.at[idx_v.at[j]], buf, sem)

        def scatter(j, buf, sem):
            dst = emb_out.at[pl.ds(base + j * chunk, chunk)]
            return pltpu.make_async_copy(buf, dst, sem)

        gather(0, buf_a, gsa).start()
        gather(0, buf_a, gsa).wait()
        gather(1, buf_b, gsb).start()
        scatter(0, buf_a, ssa).start()

        def pipe(i, carry):
            c = 2 * i + 1
            gather(c, buf_b, gsb).wait()
            scatter(c - 1, buf_a, ssa).wait()
            gather(c + 1, buf_a, gsa).start()
            scatter(c, buf_b, ssb).start()
            gather(c + 1, buf_a, gsa).wait()
            scatter(c, buf_b, ssb).wait()
            gather(c + 2, buf_b, gsb).start()
            scatter(c + 1, buf_a, ssa).start()
            return carry

        lax.fori_loop(0, n_chunks // 2 - 1, pipe, 0)

        last = n_chunks - 1
        gather(last, buf_b, gsb).wait()
        scatter(last - 1, buf_a, ssa).wait()
        scatter(last, buf_b, ssb).start()

        def ts_body(i, carry):
            p = lax.iota(jnp.int32, _LANES) + i * _LANES
            val = lax.rem(p, seq_len).astype(jnp.float32) * inv
            ts_v[pl.ds(i * _LANES, _LANES)] = val
            return carry

        lax.fori_loop(0, ts_steps, ts_body, 0)
        pltpu.sync_copy(ts_v, ts_out.at[pl.ds(base, b_per_w)])
        scatter(last, buf_b, ssb).wait()

    return sc_gather


def kernel(input_ids, table):
    b, l = input_ids.shape
    vocab, d = table.shape
    n_rows = b * l
    nw = 32
    ids3d = input_ids.reshape(-1).astype(jnp.int32).reshape(nw, -1, _CHUNK)
    emb_flat, ts_flat = _build_sc_gather(n_rows, vocab, d, l)(ids3d, table)
    return emb_flat.reshape(b, l, d), ts_flat.reshape(b, l)
